# baseline (speedup 1.0000x reference)
import functools

import jax
import jax.numpy as jnp
from jax import lax
from jax.experimental import pallas as pl
from jax.experimental.pallas import tpu as pltpu
from jax.experimental.pallas import tpu_sc as plsc

_LANES = 128


def _sc_gather(user_idx, item_idx, t_um, t_im, t_uf, t_if):
    B = user_idx.shape[0]
    info = plsc.get_sparse_core_info()
    nc, ns = info.num_cores, info.num_subcores
    nw = nc * ns
    bpw = B // nw
    sub = 128
    nsub = bpw // sub

    mesh = plsc.VectorSubcoreMesh(core_axis_name="c", subcore_axis_name="s")
    out_sds = jax.ShapeDtypeStruct((B, _LANES), jnp.float32)

    @functools.partial(
        pl.kernel,
        mesh=mesh,
        out_type=[out_sds, out_sds, out_sds, out_sds],
        scratch_types=[
            pltpu.VMEM((bpw,), jnp.int32),
            pltpu.VMEM((bpw,), jnp.int32),
            pltpu.VMEM((bpw,), jnp.int32),
            pltpu.VMEM((bpw,), jnp.int32),
            pltpu.VMEM((bpw,), jnp.int32),
            pltpu.VMEM((bpw,), jnp.int32),
            pltpu.VMEM((sub, _LANES), jnp.float32),
            pltpu.VMEM((sub, _LANES), jnp.float32),
            pltpu.VMEM((sub, _LANES), jnp.float32),
            pltpu.VMEM((sub, _LANES), jnp.float32),
            pltpu.SemaphoreType.DMA,
        ],
        compiler_params=pltpu.CompilerParams(use_tc_tiling_on_sc=True),
    )
    def gather_kernel(uidx_hbm, iidx_hbm, um_hbm, im_hbm, uf_hbm, if_hbm,
                      o_um, o_im, o_uf, o_if,
                      idx_u, idx_i, q_u, q_i, q8_u, q8_i,
                      b_um, b_im, b_uf, b_if, sem):
        wid = lax.axis_index("s") * nc + lax.axis_index("c")
        base = wid * bpw
        pltpu.sync_copy(uidx_hbm.at[pl.ds(base, bpw)], idx_u)
        pltpu.sync_copy(iidx_hbm.at[pl.ds(base, bpw)], idx_i)
        for c in range(bpw // 16):
            sl = pl.ds(c * 16, 16)
            u = idx_u[sl]
            i = idx_i[sl]
            q_u[sl] = lax.shift_right_logical(u, 2)
            q_i[sl] = lax.shift_right_logical(i, 2)
            q8_u[sl] = lax.shift_right_logical(u, 3)
            q8_i[sl] = lax.shift_right_logical(i, 3)
        for s in range(nsub):
            qsl = pl.ds(s * sub, sub)
            c1 = pltpu.async_copy(um_hbm.at[q_u.at[qsl]], b_um, sem)
            c2 = pltpu.async_copy(im_hbm.at[q_i.at[qsl]], b_im, sem)
            c3 = pltpu.async_copy(uf_hbm.at[q8_u.at[qsl]], b_uf, sem)
            c4 = pltpu.async_copy(if_hbm.at[q8_i.at[qsl]], b_if, sem)
            c1.wait()
            c2.wait()
            c3.wait()
            c4.wait()
            osl = pl.ds(base + s * sub, sub)
            pltpu.sync_copy(b_um, o_um.at[osl])
            pltpu.sync_copy(b_im, o_im.at[osl])
            pltpu.sync_copy(b_uf, o_uf.at[osl])
            pltpu.sync_copy(b_if, o_if.at[osl])

    return gather_kernel(user_idx, item_idx, t_um, t_im, t_uf, t_if)


def _mlp_body(uidx, iidx, xu, xi, xuf, xif,
              w0u, w0i, b0, w1, b1, w2, wu, wi, bo, out):
    pu = uidx[...] & 3
    pi = iidx[...] & 3
    pu8 = uidx[...] & 7
    pi8 = iidx[...] & 7

    yu = xu[...] @ w0u[...]
    yi = xi[...] @ w0i[...]
    d = w1.shape[0]
    h = jnp.zeros_like(yu[:, :d])
    for p in range(4):
        selu = (pu == p).astype(jnp.float32)
        seli = (pi == p).astype(jnp.float32)
        h = h + yu[:, p * d:(p + 1) * d] * selu + yi[:, p * d:(p + 1) * d] * seli
    h = jnp.maximum(h + b0[...], 0.0)
    h = jnp.maximum(h @ w1[...] + b1[...], 0.0)

    zu = xuf[...] @ wu[...]
    zi = xif[...] @ wi[...]
    iota8 = lax.broadcasted_iota(jnp.int32, zu.shape, 1)
    zu = jnp.sum(zu * (iota8 == pu8).astype(jnp.float32), axis=1, keepdims=True)
    zi = jnp.sum(zi * (iota8 == pi8).astype(jnp.float32), axis=1, keepdims=True)

    logit = h @ w2[...] + zu + zi + bo[...]
    out[...] = jax.nn.sigmoid(logit)


def _tc_mlp(uidx2, iidx2, xu, xi, xuf, xif, W0, b0, W1, b1, W_out, b_out):
    B = xu.shape[0]
    d_mlp = W0.shape[0] // 2
    h1 = W0.shape[1]
    h2 = W1.shape[1]
    d_mf = (W_out.shape[0] - h2) // 2
    blk = 2048

    eye4 = jnp.eye(4, dtype=jnp.float32)
    eye8 = jnp.eye(8, dtype=jnp.float32)
    w0u = jnp.kron(eye4, W0[:d_mlp])
    w0i = jnp.kron(eye4, W0[d_mlp:])
    wu = jnp.kron(eye8, W_out[h2:h2 + d_mf])
    wi = jnp.kron(eye8, W_out[h2 + d_mf:])
    w2 = W_out[:h2]
    b0r = b0.reshape(1, h1)
    b1r = b1.reshape(1, h2)
    bor = b_out.reshape(1, 1)

    full = lambda a: pl.BlockSpec(a.shape, lambda i: (0,) * a.ndim)
    bspec = lambda d: pl.BlockSpec((blk, d), lambda i: (i, 0))

    args = (uidx2, iidx2, xu, xi, xuf, xif,
            w0u, w0i, b0r, W1, b1r, w2, wu, wi, bor)
    specs = [bspec(1), bspec(1), bspec(_LANES), bspec(_LANES),
             bspec(_LANES), bspec(_LANES)] + [full(a) for a in args[6:]]

    return pl.pallas_call(
        _mlp_body,
        grid=(B // blk,),
        in_specs=specs,
        out_specs=pl.BlockSpec((blk, 1), lambda i: (i, 0)),
        out_shape=jax.ShapeDtypeStruct((B, 1), jnp.float32),
    )(*args)


def kernel(user_indices, item_indices, emb_user_mlp, emb_item_mlp,
           emb_user_mf, emb_item_mf, W0, b0, W1, b1, W_out, b_out):
    nu, d_mlp = emb_user_mlp.shape
    ni = emb_item_mlp.shape[0]
    d_mf = emb_user_mf.shape[1]
    t_um = emb_user_mlp.reshape(nu * d_mlp // _LANES, _LANES)
    t_im = emb_item_mlp.reshape(ni * d_mlp // _LANES, _LANES)
    t_uf = emb_user_mf.reshape(nu * d_mf // _LANES, _LANES)
    t_if = emb_item_mf.reshape(ni * d_mf // _LANES, _LANES)

    xu, xi, xuf, xif = _sc_gather(
        user_indices, item_indices, t_um, t_im, t_uf, t_if)
    uidx2 = user_indices.reshape(-1, 1)
    iidx2 = item_indices.reshape(-1, 1)
    return _tc_mlp(uidx2, iidx2, xu, xi, xuf, xif,
                   W0, b0, W1, b1, W_out, b_out)


# speedup vs baseline: 1.3046x; 1.3046x over previous
import functools

import jax
import jax.numpy as jnp
from jax import lax
from jax.experimental import pallas as pl
from jax.experimental.pallas import tpu as pltpu
from jax.experimental.pallas import tpu_sc as plsc

_LANES = 128


def _repack_body(n, xt, out):
    d, w = xt.shape
    g = _LANES // d
    wc = w // g
    col = lax.broadcasted_iota(jnp.int32, (d, w), 1)
    x = jnp.where(col < n - pl.program_id(0) * w, xt[...], 0.0)
    out[...] = jnp.concatenate(
        [x[:, p * wc:(p + 1) * wc].T for p in range(g)], axis=1)


def _repack(xT, blk_n):
    d, n = xT.shape
    grid = (n + blk_n - 1) // blk_n
    rows_per_blk = blk_n * d // _LANES
    return pl.pallas_call(
        functools.partial(_repack_body, n),
        grid=(grid,),
        in_specs=[pl.BlockSpec((d, blk_n), lambda i: (0, i))],
        out_specs=pl.BlockSpec((rows_per_blk, _LANES), lambda i: (i, 0)),
        out_shape=jax.ShapeDtypeStruct((grid * rows_per_blk, _LANES),
                                       jnp.float32),
    )(xT)


def _sc_gather(user_idx, item_idx, t_um, t_im, t_uf, t_if):
    B = user_idx.shape[0]
    info = plsc.get_sparse_core_info()
    nc, ns = info.num_cores, info.num_subcores
    nw = nc * ns
    bpw = B // nw
    sub = 128
    nsub = bpw // sub

    mesh = plsc.VectorSubcoreMesh(core_axis_name="c", subcore_axis_name="s")
    out_sds = jax.ShapeDtypeStruct((B, _LANES), jnp.float32)

    @functools.partial(
        pl.kernel,
        mesh=mesh,
        out_type=[out_sds, out_sds, out_sds, out_sds],
        scratch_types=[
            pltpu.VMEM((bpw,), jnp.int32),
            pltpu.VMEM((bpw,), jnp.int32),
            pltpu.VMEM((bpw,), jnp.int32),
            pltpu.VMEM((bpw,), jnp.int32),
            pltpu.VMEM((bpw,), jnp.int32),
            pltpu.VMEM((bpw,), jnp.int32),
            pltpu.VMEM((sub, _LANES), jnp.float32),
            pltpu.VMEM((sub, _LANES), jnp.float32),
            pltpu.VMEM((sub, _LANES), jnp.float32),
            pltpu.VMEM((sub, _LANES), jnp.float32),
            pltpu.SemaphoreType.DMA,
        ],
        compiler_params=pltpu.CompilerParams(use_tc_tiling_on_sc=True),
    )
    def gather_kernel(uidx_hbm, iidx_hbm, um_hbm, im_hbm, uf_hbm, if_hbm,
                      o_um, o_im, o_uf, o_if,
                      idx_u, idx_i, q_u, q_i, q8_u, q8_i,
                      b_um, b_im, b_uf, b_if, sem):
        wid = lax.axis_index("s") * nc + lax.axis_index("c")
        base = wid * bpw
        pltpu.sync_copy(uidx_hbm.at[pl.ds(base, bpw)], idx_u)
        pltpu.sync_copy(iidx_hbm.at[pl.ds(base, bpw)], idx_i)
        for c in range(bpw // 16):
            sl = pl.ds(c * 16, 16)
            u = idx_u[sl]
            i = idx_i[sl]
            ub = lax.shift_right_logical(u, 12)
            ib = lax.shift_right_logical(i, 12)
            q_u[sl] = lax.shift_left(ub, 10) | (u & 1023)
            q_i[sl] = lax.shift_left(ib, 10) | (i & 1023)
            q8_u[sl] = lax.shift_left(ub, 9) | (u & 511)
            q8_i[sl] = lax.shift_left(ib, 9) | (i & 511)
        for s in range(nsub):
            qsl = pl.ds(s * sub, sub)
            c1 = pltpu.async_copy(um_hbm.at[q_u.at[qsl]], b_um, sem)
            c2 = pltpu.async_copy(im_hbm.at[q_i.at[qsl]], b_im, sem)
            c3 = pltpu.async_copy(uf_hbm.at[q8_u.at[qsl]], b_uf, sem)
            c4 = pltpu.async_copy(if_hbm.at[q8_i.at[qsl]], b_if, sem)
            c1.wait()
            c2.wait()
            c3.wait()
            c4.wait()
            osl = pl.ds(base + s * sub, sub)
            pltpu.sync_copy(b_um, o_um.at[osl])
            pltpu.sync_copy(b_im, o_im.at[osl])
            pltpu.sync_copy(b_uf, o_uf.at[osl])
            pltpu.sync_copy(b_if, o_if.at[osl])

    return gather_kernel(user_idx, item_idx, t_um, t_im, t_uf, t_if)


def _mlp_body(uidx, iidx, xu, xi, xuf, xif,
              w0u, w0i, b0, w1, b1, w2, wu, wi, bo, out):
    pu = lax.shift_right_logical(uidx[...], 10) & 3
    pi = lax.shift_right_logical(iidx[...], 10) & 3
    pu8 = lax.shift_right_logical(uidx[...], 9) & 7
    pi8 = lax.shift_right_logical(iidx[...], 9) & 7

    yu = xu[...] @ w0u[...]
    yi = xi[...] @ w0i[...]
    d = w1.shape[0]
    h = jnp.zeros_like(yu[:, :d])
    for p in range(4):
        h = (h + jnp.where(pu == p, yu[:, p * d:(p + 1) * d], 0.0)
             + jnp.where(pi == p, yi[:, p * d:(p + 1) * d], 0.0))
    h = jnp.maximum(h + b0[...], 0.0)
    h = jnp.maximum(h @ w1[...] + b1[...], 0.0)

    zu = xuf[...] @ wu[...]
    zi = xif[...] @ wi[...]
    iota8 = lax.broadcasted_iota(jnp.int32, zu.shape, 1)
    zu = jnp.sum(jnp.where(iota8 == pu8, zu, 0.0), axis=1, keepdims=True)
    zi = jnp.sum(jnp.where(iota8 == pi8, zi, 0.0), axis=1, keepdims=True)

    logit = h @ w2[...] + zu + zi + bo[...]
    out[...] = jax.nn.sigmoid(logit)


def _tc_mlp(uidx2, iidx2, xu, xi, xuf, xif, W0, b0, W1, b1, W_out, b_out):
    B = xu.shape[0]
    d_mlp = W0.shape[0] // 2
    h1 = W0.shape[1]
    h2 = W1.shape[1]
    d_mf = (W_out.shape[0] - h2) // 2
    blk = 2048

    eye4 = jnp.eye(4, dtype=jnp.float32)
    eye8 = jnp.eye(8, dtype=jnp.float32)
    w0u = jnp.kron(eye4, W0[:d_mlp])
    w0i = jnp.kron(eye4, W0[d_mlp:])
    wu = jnp.kron(eye8, W_out[h2:h2 + d_mf])
    wi = jnp.kron(eye8, W_out[h2 + d_mf:])
    w2 = W_out[:h2]
    b0r = b0.reshape(1, h1)
    b1r = b1.reshape(1, h2)
    bor = b_out.reshape(1, 1)

    full = lambda a: pl.BlockSpec(a.shape, lambda i: (0,) * a.ndim)
    bspec = lambda d: pl.BlockSpec((blk, d), lambda i: (i, 0))

    args = (uidx2, iidx2, xu, xi, xuf, xif,
            w0u, w0i, b0r, W1, b1r, w2, wu, wi, bor)
    specs = [bspec(1), bspec(1), bspec(_LANES), bspec(_LANES),
             bspec(_LANES), bspec(_LANES)] + [full(a) for a in args[6:]]

    return pl.pallas_call(
        _mlp_body,
        grid=(B // blk,),
        in_specs=specs,
        out_specs=pl.BlockSpec((blk, 1), lambda i: (i, 0)),
        out_shape=jax.ShapeDtypeStruct((B, 1), jnp.float32),
    )(*args)


def kernel(user_indices, item_indices, emb_user_mlp, emb_item_mlp,
           emb_user_mf, emb_item_mf, W0, b0, W1, b1, W_out, b_out):
    t_um = _repack(emb_user_mlp.T, 4096)
    t_im = _repack(emb_item_mlp.T, 4096)
    t_uf = _repack(emb_user_mf.T, 4096)
    t_if = _repack(emb_item_mf.T, 4096)

    xu, xi, xuf, xif = _sc_gather(
        user_indices, item_indices, t_um, t_im, t_uf, t_if)
    uidx2 = user_indices.reshape(-1, 1)
    iidx2 = item_indices.reshape(-1, 1)
    return _tc_mlp(uidx2, iidx2, xu, xi, xuf, xif,
                   W0, b0, W1, b1, W_out, b_out)


# speedup vs baseline: 1.8551x; 1.4220x over previous
import functools

import jax
import jax.numpy as jnp
from jax import lax
from jax.experimental import pallas as pl
from jax.experimental.pallas import tpu as pltpu
from jax.experimental.pallas import tpu_sc as plsc

_LANES = 128


def _repack_body(n, xt, out):
    d, w = xt.shape
    g = _LANES // d
    wc = w // g
    col = lax.broadcasted_iota(jnp.int32, (d, w), 1)
    x = jnp.where(col < n - pl.program_id(0) * w, xt[...], 0.0)
    lhs = jnp.concatenate([x[:, p * wc:(p + 1) * wc] for p in range(g)],
                          axis=0)
    eye = jnp.eye(_LANES, dtype=jnp.float32)
    out[...] = lax.dot_general(lhs, eye, (((0,), (0,)), ((), ())),
                               preferred_element_type=jnp.float32)


def _repack(xT, blk_n):
    d, n = xT.shape
    grid = (n + blk_n - 1) // blk_n
    rows_per_blk = blk_n * d // _LANES
    return pl.pallas_call(
        functools.partial(_repack_body, n),
        grid=(grid,),
        in_specs=[pl.BlockSpec((d, blk_n), lambda i: (0, i))],
        out_specs=pl.BlockSpec((rows_per_blk, _LANES), lambda i: (i, 0)),
        out_shape=jax.ShapeDtypeStruct((grid * rows_per_blk, _LANES),
                                       jnp.float32),
    )(xT)


def _sc_gather(user_idx, item_idx, t_um, t_im, t_uf, t_if):
    B = user_idx.shape[0]
    info = plsc.get_sparse_core_info()
    nc, ns = info.num_cores, info.num_subcores
    nw = nc * ns
    bpw = B // nw
    sub = 128
    nsub = bpw // sub

    mesh = plsc.VectorSubcoreMesh(core_axis_name="c", subcore_axis_name="s")
    out_sds = jax.ShapeDtypeStruct((B, _LANES), jnp.float32)

    @functools.partial(
        pl.kernel,
        mesh=mesh,
        out_type=[out_sds, out_sds, out_sds, out_sds],
        scratch_types=[
            pltpu.VMEM((bpw,), jnp.int32),
            pltpu.VMEM((bpw,), jnp.int32),
            pltpu.VMEM((bpw,), jnp.int32),
            pltpu.VMEM((bpw,), jnp.int32),
            pltpu.VMEM((bpw,), jnp.int32),
            pltpu.VMEM((bpw,), jnp.int32),
            pltpu.VMEM((sub, _LANES), jnp.float32),
            pltpu.VMEM((sub, _LANES), jnp.float32),
            pltpu.VMEM((sub, _LANES), jnp.float32),
            pltpu.VMEM((sub, _LANES), jnp.float32),
            pltpu.SemaphoreType.DMA,
        ],
        compiler_params=pltpu.CompilerParams(use_tc_tiling_on_sc=True),
    )
    def gather_kernel(uidx_hbm, iidx_hbm, um_hbm, im_hbm, uf_hbm, if_hbm,
                      o_um, o_im, o_uf, o_if,
                      idx_u, idx_i, q_u, q_i, q8_u, q8_i,
                      b_um, b_im, b_uf, b_if, sem):
        wid = lax.axis_index("s") * nc + lax.axis_index("c")
        base = wid * bpw
        pltpu.sync_copy(uidx_hbm.at[pl.ds(base, bpw)], idx_u)
        pltpu.sync_copy(iidx_hbm.at[pl.ds(base, bpw)], idx_i)
        for c in range(bpw // 16):
            sl = pl.ds(c * 16, 16)
            u = idx_u[sl]
            i = idx_i[sl]
            ub = lax.shift_right_logical(u, 12)
            ib = lax.shift_right_logical(i, 12)
            q_u[sl] = lax.shift_left(ub, 10) | (u & 1023)
            q_i[sl] = lax.shift_left(ib, 10) | (i & 1023)
            q8_u[sl] = lax.shift_left(ub, 9) | (u & 511)
            q8_i[sl] = lax.shift_left(ib, 9) | (i & 511)
        for s in range(nsub):
            qsl = pl.ds(s * sub, sub)
            c1 = pltpu.async_copy(um_hbm.at[q_u.at[qsl]], b_um, sem)
            c2 = pltpu.async_copy(im_hbm.at[q_i.at[qsl]], b_im, sem)
            c3 = pltpu.async_copy(uf_hbm.at[q8_u.at[qsl]], b_uf, sem)
            c4 = pltpu.async_copy(if_hbm.at[q8_i.at[qsl]], b_if, sem)
            c1.wait()
            c2.wait()
            c3.wait()
            c4.wait()
            osl = pl.ds(base + s * sub, sub)
            pltpu.sync_copy(b_um, o_um.at[osl])
            pltpu.sync_copy(b_im, o_im.at[osl])
            pltpu.sync_copy(b_uf, o_uf.at[osl])
            pltpu.sync_copy(b_if, o_if.at[osl])

    return gather_kernel(user_idx, item_idx, t_um, t_im, t_uf, t_if)


def _mlp_body(uidx, iidx, xu, xi, xuf, xif,
              w0u, w0i, b0, w1, b1, w2, wu, wi, bo, out):
    pu = lax.shift_right_logical(uidx[...], 10) & 3
    pi = lax.shift_right_logical(iidx[...], 10) & 3
    pu8 = lax.shift_right_logical(uidx[...], 9) & 7
    pi8 = lax.shift_right_logical(iidx[...], 9) & 7

    xuv = xu[...]
    xiv = xi[...]
    xuv = jnp.where(jnp.isfinite(xuv), xuv, 0.0)
    xiv = jnp.where(jnp.isfinite(xiv), xiv, 0.0)
    yu = xuv @ w0u[...]
    yi = xiv @ w0i[...]
    d = w1.shape[0]
    h = jnp.zeros_like(yu[:, :d])
    for p in range(4):
        h = (h + jnp.where(pu == p, yu[:, p * d:(p + 1) * d], 0.0)
             + jnp.where(pi == p, yi[:, p * d:(p + 1) * d], 0.0))
    h = jnp.maximum(h + b0[...], 0.0)
    h = jnp.maximum(h @ w1[...] + b1[...], 0.0)

    xufv = xuf[...]
    xifv = xif[...]
    xufv = jnp.where(jnp.isfinite(xufv), xufv, 0.0)
    xifv = jnp.where(jnp.isfinite(xifv), xifv, 0.0)
    zu = xufv @ wu[...]
    zi = xifv @ wi[...]
    iota8 = lax.broadcasted_iota(jnp.int32, zu.shape, 1)
    zu = jnp.sum(jnp.where(iota8 == pu8, zu, 0.0), axis=1, keepdims=True)
    zi = jnp.sum(jnp.where(iota8 == pi8, zi, 0.0), axis=1, keepdims=True)

    logit = h @ w2[...] + zu + zi + bo[...]
    out[...] = jax.nn.sigmoid(logit)


def _tc_mlp(uidx2, iidx2, xu, xi, xuf, xif, W0, b0, W1, b1, W_out, b_out):
    B = xu.shape[0]
    d_mlp = W0.shape[0] // 2
    h1 = W0.shape[1]
    h2 = W1.shape[1]
    d_mf = (W_out.shape[0] - h2) // 2
    blk = 2048

    eye4 = jnp.eye(4, dtype=jnp.float32)
    eye8 = jnp.eye(8, dtype=jnp.float32)
    w0u = jnp.kron(eye4, W0[:d_mlp])
    w0i = jnp.kron(eye4, W0[d_mlp:])
    wu = jnp.kron(eye8, W_out[h2:h2 + d_mf])
    wi = jnp.kron(eye8, W_out[h2 + d_mf:])
    w2 = W_out[:h2]
    b0r = b0.reshape(1, h1)
    b1r = b1.reshape(1, h2)
    bor = b_out.reshape(1, 1)

    full = lambda a: pl.BlockSpec(a.shape, lambda i: (0,) * a.ndim)
    bspec = lambda d: pl.BlockSpec((blk, d), lambda i: (i, 0))

    args = (uidx2, iidx2, xu, xi, xuf, xif,
            w0u, w0i, b0r, W1, b1r, w2, wu, wi, bor)
    specs = [bspec(1), bspec(1), bspec(_LANES), bspec(_LANES),
             bspec(_LANES), bspec(_LANES)] + [full(a) for a in args[6:]]

    return pl.pallas_call(
        _mlp_body,
        grid=(B // blk,),
        in_specs=specs,
        out_specs=pl.BlockSpec((blk, 1), lambda i: (i, 0)),
        out_shape=jax.ShapeDtypeStruct((B, 1), jnp.float32),
    )(*args)


def kernel(user_indices, item_indices, emb_user_mlp, emb_item_mlp,
           emb_user_mf, emb_item_mf, W0, b0, W1, b1, W_out, b_out):
    t_um = _repack(emb_user_mlp.T, 4096)
    t_im = _repack(emb_item_mlp.T, 4096)
    t_uf = _repack(emb_user_mf.T, 4096)
    t_if = _repack(emb_item_mf.T, 4096)

    xu, xi, xuf, xif = _sc_gather(
        user_indices, item_indices, t_um, t_im, t_uf, t_if)
    uidx2 = user_indices.reshape(-1, 1)
    iidx2 = item_indices.reshape(-1, 1)
    return _tc_mlp(uidx2, iidx2, xu, xi, xuf, xif,
                   W0, b0, W1, b1, W_out, b_out)


# speedup vs baseline: 2.5793x; 1.3904x over previous
import functools

import jax
import jax.numpy as jnp
from jax import lax
from jax.experimental import pallas as pl
from jax.experimental.pallas import tpu as pltpu
from jax.experimental.pallas import tpu_sc as plsc

_LANES = 128
_BLOG = 13
_QLOG_MLP = _BLOG - 2
_QLOG_MF = _BLOG - 3


def _repack_body(n, xt, out):
    d, w = xt.shape
    g = _LANES // d
    wc = w // g
    col = lax.broadcasted_iota(jnp.int32, (d, w), 1)
    x = jnp.where(col < n - pl.program_id(0) * w, xt[...], 0.0)
    lhs = jnp.concatenate([x[:, p * wc:(p + 1) * wc] for p in range(g)],
                          axis=0)
    eye = jnp.eye(_LANES, dtype=jnp.float32)
    out[...] = lax.dot_general(lhs, eye, (((0,), (0,)), ((), ())),
                               preferred_element_type=jnp.float32)


def _repack(xT, blk_n):
    d, n = xT.shape
    grid = (n + blk_n - 1) // blk_n
    rows_per_blk = blk_n * d // _LANES
    return pl.pallas_call(
        functools.partial(_repack_body, n),
        grid=(grid,),
        in_specs=[pl.BlockSpec((d, blk_n), lambda i: (0, i))],
        out_specs=pl.BlockSpec((rows_per_blk, _LANES), lambda i: (i, 0)),
        out_shape=jax.ShapeDtypeStruct((grid * rows_per_blk, _LANES),
                                       jnp.float32),
    )(xT)


def _sc_gather(user_idx, item_idx, t_um, t_im, t_uf, t_if):
    B = user_idx.shape[0]
    info = plsc.get_sparse_core_info()
    nc, ns = info.num_cores, info.num_subcores
    nw = nc * ns
    bpw = B // nw
    sub = 128
    nsub = bpw // sub

    mesh = plsc.VectorSubcoreMesh(core_axis_name="c", subcore_axis_name="s")
    out_sds = jax.ShapeDtypeStruct((B, _LANES), jnp.float32)

    @functools.partial(
        pl.kernel,
        mesh=mesh,
        out_type=[out_sds, out_sds, out_sds, out_sds],
        scratch_types=[
            pltpu.VMEM((bpw,), jnp.int32),
            pltpu.VMEM((bpw,), jnp.int32),
            pltpu.VMEM((bpw,), jnp.int32),
            pltpu.VMEM((bpw,), jnp.int32),
            pltpu.VMEM((bpw,), jnp.int32),
            pltpu.VMEM((bpw,), jnp.int32),
            pltpu.VMEM((sub, _LANES), jnp.float32),
            pltpu.VMEM((sub, _LANES), jnp.float32),
            pltpu.VMEM((sub, _LANES), jnp.float32),
            pltpu.VMEM((sub, _LANES), jnp.float32),
            pltpu.SemaphoreType.DMA,
        ],
        compiler_params=pltpu.CompilerParams(use_tc_tiling_on_sc=True),
    )
    def gather_kernel(uidx_hbm, iidx_hbm, um_hbm, im_hbm, uf_hbm, if_hbm,
                      o_um, o_im, o_uf, o_if,
                      idx_u, idx_i, q_u, q_i, q8_u, q8_i,
                      b_um, b_im, b_uf, b_if, sem):
        wid = lax.axis_index("s") * nc + lax.axis_index("c")
        base = wid * bpw
        pltpu.sync_copy(uidx_hbm.at[pl.ds(base, bpw)], idx_u)
        pltpu.sync_copy(iidx_hbm.at[pl.ds(base, bpw)], idx_i)
        for c in range(bpw // 16):
            sl = pl.ds(c * 16, 16)
            u = idx_u[sl]
            i = idx_i[sl]
            ub = lax.shift_right_logical(u, _BLOG)
            ib = lax.shift_right_logical(i, _BLOG)
            q_u[sl] = lax.shift_left(ub, _QLOG_MLP) | (u & (2**_QLOG_MLP - 1))
            q_i[sl] = lax.shift_left(ib, _QLOG_MLP) | (i & (2**_QLOG_MLP - 1))
            q8_u[sl] = lax.shift_left(ub, _QLOG_MF) | (u & (2**_QLOG_MF - 1))
            q8_i[sl] = lax.shift_left(ib, _QLOG_MF) | (i & (2**_QLOG_MF - 1))
        for s in range(nsub):
            qsl = pl.ds(s * sub, sub)
            c1 = pltpu.async_copy(um_hbm.at[q_u.at[qsl]], b_um, sem)
            c2 = pltpu.async_copy(im_hbm.at[q_i.at[qsl]], b_im, sem)
            c3 = pltpu.async_copy(uf_hbm.at[q8_u.at[qsl]], b_uf, sem)
            c4 = pltpu.async_copy(if_hbm.at[q8_i.at[qsl]], b_if, sem)
            c1.wait()
            c2.wait()
            c3.wait()
            c4.wait()
            osl = pl.ds(base + s * sub, sub)
            pltpu.sync_copy(b_um, o_um.at[osl])
            pltpu.sync_copy(b_im, o_im.at[osl])
            pltpu.sync_copy(b_uf, o_uf.at[osl])
            pltpu.sync_copy(b_if, o_if.at[osl])

    return gather_kernel(user_idx, item_idx, t_um, t_im, t_uf, t_if)


def _mlp_body(uidx, iidx, xu, xi, xuf, xif,
              w0u, w0i, b0, w1, b1, w2, wu, wi, bo, out):
    pu = lax.shift_right_logical(uidx[...], _QLOG_MLP) & 3
    pi = lax.shift_right_logical(iidx[...], _QLOG_MLP) & 3
    pu8 = lax.shift_right_logical(uidx[...], _QLOG_MF) & 7
    pi8 = lax.shift_right_logical(iidx[...], _QLOG_MF) & 7

    xuv = xu[...]
    xiv = xi[...]
    xuv = jnp.where(jnp.isfinite(xuv), xuv, 0.0)
    xiv = jnp.where(jnp.isfinite(xiv), xiv, 0.0)
    yu = xuv @ w0u[...]
    yi = xiv @ w0i[...]
    d = w1.shape[0]
    h = jnp.zeros_like(yu[:, :d])
    for p in range(4):
        h = (h + jnp.where(pu == p, yu[:, p * d:(p + 1) * d], 0.0)
             + jnp.where(pi == p, yi[:, p * d:(p + 1) * d], 0.0))
    h = jnp.maximum(h + b0[...], 0.0)
    h = jnp.maximum(h @ w1[...] + b1[...], 0.0)

    xufv = xuf[...]
    xifv = xif[...]
    xufv = jnp.where(jnp.isfinite(xufv), xufv, 0.0)
    xifv = jnp.where(jnp.isfinite(xifv), xifv, 0.0)
    zu = xufv @ wu[...]
    zi = xifv @ wi[...]
    iota8 = lax.broadcasted_iota(jnp.int32, zu.shape, 1)
    zu = jnp.sum(jnp.where(iota8 == pu8, zu, 0.0), axis=1, keepdims=True)
    zi = jnp.sum(jnp.where(iota8 == pi8, zi, 0.0), axis=1, keepdims=True)

    logit = h @ w2[...] + zu + zi + bo[...]
    out[...] = jax.nn.sigmoid(logit)


def _tc_mlp(uidx2, iidx2, xu, xi, xuf, xif, W0, b0, W1, b1, W_out, b_out):
    B = xu.shape[0]
    d_mlp = W0.shape[0] // 2
    h1 = W0.shape[1]
    h2 = W1.shape[1]
    d_mf = (W_out.shape[0] - h2) // 2
    blk = 2048

    eye4 = jnp.eye(4, dtype=jnp.float32)
    eye8 = jnp.eye(8, dtype=jnp.float32)
    w0u = jnp.kron(eye4, W0[:d_mlp])
    w0i = jnp.kron(eye4, W0[d_mlp:])
    wu = jnp.kron(eye8, W_out[h2:h2 + d_mf])
    wi = jnp.kron(eye8, W_out[h2 + d_mf:])
    w2 = W_out[:h2]
    b0r = b0.reshape(1, h1)
    b1r = b1.reshape(1, h2)
    bor = b_out.reshape(1, 1)

    full = lambda a: pl.BlockSpec(a.shape, lambda i: (0,) * a.ndim)
    bspec = lambda d: pl.BlockSpec((blk, d), lambda i: (i, 0))

    args = (uidx2, iidx2, xu, xi, xuf, xif,
            w0u, w0i, b0r, W1, b1r, w2, wu, wi, bor)
    specs = [bspec(1), bspec(1), bspec(_LANES), bspec(_LANES),
             bspec(_LANES), bspec(_LANES)] + [full(a) for a in args[6:]]

    return pl.pallas_call(
        _mlp_body,
        grid=(B // blk,),
        in_specs=specs,
        out_specs=pl.BlockSpec((blk, 1), lambda i: (i, 0)),
        out_shape=jax.ShapeDtypeStruct((B, 1), jnp.float32),
    )(*args)


def kernel(user_indices, item_indices, emb_user_mlp, emb_item_mlp,
           emb_user_mf, emb_item_mf, W0, b0, W1, b1, W_out, b_out):
    t_um = _repack(emb_user_mlp.T, 1 << _BLOG)
    t_im = _repack(emb_item_mlp.T, 1 << _BLOG)
    t_uf = _repack(emb_user_mf.T, 1 << _BLOG)
    t_if = _repack(emb_item_mf.T, 1 << _BLOG)

    xu, xi, xuf, xif = _sc_gather(
        user_indices, item_indices, t_um, t_im, t_uf, t_if)
    uidx2 = user_indices.reshape(-1, 1)
    iidx2 = item_indices.reshape(-1, 1)
    return _tc_mlp(uidx2, iidx2, xu, xi, xuf, xif,
                   W0, b0, W1, b1, W_out, b_out)


# speedup vs baseline: 3.2521x; 1.2608x over previous
import functools

import jax
import jax.numpy as jnp
from jax import lax
from jax.experimental import pallas as pl
from jax.experimental.pallas import tpu as pltpu
from jax.experimental.pallas import tpu_sc as plsc

_LANES = 128
_BLOG = 14
_QLOG_MLP = _BLOG - 2
_QLOG_MF = _BLOG - 3


def _repack_body(n, xt, out):
    d, w = xt.shape
    g = _LANES // d
    wc = w // g
    col = lax.broadcasted_iota(jnp.int32, (d, w), 1)
    x = jnp.where(col < n - pl.program_id(0) * w, xt[...], 0.0)
    lhs = jnp.concatenate([x[:, p * wc:(p + 1) * wc] for p in range(g)],
                          axis=0)
    eye = jnp.eye(_LANES, dtype=jnp.float32)
    out[...] = lax.dot_general(lhs, eye, (((0,), (0,)), ((), ())),
                               preferred_element_type=jnp.float32)


def _repack(xT, blk_n):
    d, n = xT.shape
    grid = (n + blk_n - 1) // blk_n
    rows_per_blk = blk_n * d // _LANES
    return pl.pallas_call(
        functools.partial(_repack_body, n),
        grid=(grid,),
        in_specs=[pl.BlockSpec((d, blk_n), lambda i: (0, i))],
        out_specs=pl.BlockSpec((rows_per_blk, _LANES), lambda i: (i, 0)),
        out_shape=jax.ShapeDtypeStruct((grid * rows_per_blk, _LANES),
                                       jnp.float32),
    )(xT)


def _sc_gather(user_idx, item_idx, t_um, t_im, t_uf, t_if):
    B = user_idx.shape[0]
    info = plsc.get_sparse_core_info()
    nc, ns = info.num_cores, info.num_subcores
    nw = nc * ns
    bpw = B // nw
    sub = 128
    nsub = bpw // sub

    mesh = plsc.VectorSubcoreMesh(core_axis_name="c", subcore_axis_name="s")
    out_sds = jax.ShapeDtypeStruct((B, _LANES), jnp.float32)

    @functools.partial(
        pl.kernel,
        mesh=mesh,
        out_type=[out_sds, out_sds, out_sds, out_sds],
        scratch_types=[
            pltpu.VMEM((bpw,), jnp.int32),
            pltpu.VMEM((bpw,), jnp.int32),
            pltpu.VMEM((bpw,), jnp.int32),
            pltpu.VMEM((bpw,), jnp.int32),
            pltpu.VMEM((bpw,), jnp.int32),
            pltpu.VMEM((bpw,), jnp.int32),
            pltpu.VMEM((sub, _LANES), jnp.float32),
            pltpu.VMEM((sub, _LANES), jnp.float32),
            pltpu.VMEM((sub, _LANES), jnp.float32),
            pltpu.VMEM((sub, _LANES), jnp.float32),
            pltpu.SemaphoreType.DMA,
        ],
        compiler_params=pltpu.CompilerParams(use_tc_tiling_on_sc=True),
    )
    def gather_kernel(uidx_hbm, iidx_hbm, um_hbm, im_hbm, uf_hbm, if_hbm,
                      o_um, o_im, o_uf, o_if,
                      idx_u, idx_i, q_u, q_i, q8_u, q8_i,
                      b_um, b_im, b_uf, b_if, sem):
        wid = lax.axis_index("s") * nc + lax.axis_index("c")
        base = wid * bpw
        pltpu.sync_copy(uidx_hbm.at[pl.ds(base, bpw)], idx_u)
        pltpu.sync_copy(iidx_hbm.at[pl.ds(base, bpw)], idx_i)
        for c in range(bpw // 16):
            sl = pl.ds(c * 16, 16)
            u = idx_u[sl]
            i = idx_i[sl]
            ub = lax.shift_right_logical(u, _BLOG)
            ib = lax.shift_right_logical(i, _BLOG)
            q_u[sl] = lax.shift_left(ub, _QLOG_MLP) | (u & (2**_QLOG_MLP - 1))
            q_i[sl] = lax.shift_left(ib, _QLOG_MLP) | (i & (2**_QLOG_MLP - 1))
            q8_u[sl] = lax.shift_left(ub, _QLOG_MF) | (u & (2**_QLOG_MF - 1))
            q8_i[sl] = lax.shift_left(ib, _QLOG_MF) | (i & (2**_QLOG_MF - 1))
        for s in range(nsub):
            qsl = pl.ds(s * sub, sub)
            c1 = pltpu.async_copy(um_hbm.at[q_u.at[qsl]], b_um, sem)
            c2 = pltpu.async_copy(im_hbm.at[q_i.at[qsl]], b_im, sem)
            c3 = pltpu.async_copy(uf_hbm.at[q8_u.at[qsl]], b_uf, sem)
            c4 = pltpu.async_copy(if_hbm.at[q8_i.at[qsl]], b_if, sem)
            c1.wait()
            c2.wait()
            c3.wait()
            c4.wait()
            osl = pl.ds(base + s * sub, sub)
            pltpu.sync_copy(b_um, o_um.at[osl])
            pltpu.sync_copy(b_im, o_im.at[osl])
            pltpu.sync_copy(b_uf, o_uf.at[osl])
            pltpu.sync_copy(b_if, o_if.at[osl])

    return gather_kernel(user_idx, item_idx, t_um, t_im, t_uf, t_if)


def _mlp_body(uidx, iidx, xu, xi, xuf, xif,
              w0u, w0i, b0, w1, b1, w2, wu, wi, bo, out):
    pu = lax.shift_right_logical(uidx[...], _QLOG_MLP) & 3
    pi = lax.shift_right_logical(iidx[...], _QLOG_MLP) & 3
    pu8 = lax.shift_right_logical(uidx[...], _QLOG_MF) & 7
    pi8 = lax.shift_right_logical(iidx[...], _QLOG_MF) & 7

    xuv = xu[...]
    xiv = xi[...]
    xuv = jnp.where(jnp.isfinite(xuv), xuv, 0.0)
    xiv = jnp.where(jnp.isfinite(xiv), xiv, 0.0)
    yu = xuv @ w0u[...]
    yi = xiv @ w0i[...]
    d = w1.shape[0]
    h = jnp.zeros_like(yu[:, :d])
    for p in range(4):
        h = (h + jnp.where(pu == p, yu[:, p * d:(p + 1) * d], 0.0)
             + jnp.where(pi == p, yi[:, p * d:(p + 1) * d], 0.0))
    h = jnp.maximum(h + b0[...], 0.0)
    h = jnp.maximum(h @ w1[...] + b1[...], 0.0)

    xufv = xuf[...]
    xifv = xif[...]
    xufv = jnp.where(jnp.isfinite(xufv), xufv, 0.0)
    xifv = jnp.where(jnp.isfinite(xifv), xifv, 0.0)
    zu = xufv @ wu[...]
    zi = xifv @ wi[...]
    iota8 = lax.broadcasted_iota(jnp.int32, zu.shape, 1)
    zu = jnp.sum(jnp.where(iota8 == pu8, zu, 0.0), axis=1, keepdims=True)
    zi = jnp.sum(jnp.where(iota8 == pi8, zi, 0.0), axis=1, keepdims=True)

    logit = h @ w2[...] + zu + zi + bo[...]
    out[...] = jax.nn.sigmoid(logit)


def _tc_mlp(uidx2, iidx2, xu, xi, xuf, xif, W0, b0, W1, b1, W_out, b_out):
    B = xu.shape[0]
    d_mlp = W0.shape[0] // 2
    h1 = W0.shape[1]
    h2 = W1.shape[1]
    d_mf = (W_out.shape[0] - h2) // 2
    blk = 2048

    eye4 = jnp.eye(4, dtype=jnp.float32)
    eye8 = jnp.eye(8, dtype=jnp.float32)
    w0u = jnp.kron(eye4, W0[:d_mlp])
    w0i = jnp.kron(eye4, W0[d_mlp:])
    wu = jnp.kron(eye8, W_out[h2:h2 + d_mf])
    wi = jnp.kron(eye8, W_out[h2 + d_mf:])
    w2 = W_out[:h2]
    b0r = b0.reshape(1, h1)
    b1r = b1.reshape(1, h2)
    bor = b_out.reshape(1, 1)

    full = lambda a: pl.BlockSpec(a.shape, lambda i: (0,) * a.ndim)
    bspec = lambda d: pl.BlockSpec((blk, d), lambda i: (i, 0))

    args = (uidx2, iidx2, xu, xi, xuf, xif,
            w0u, w0i, b0r, W1, b1r, w2, wu, wi, bor)
    specs = [bspec(1), bspec(1), bspec(_LANES), bspec(_LANES),
             bspec(_LANES), bspec(_LANES)] + [full(a) for a in args[6:]]

    return pl.pallas_call(
        _mlp_body,
        grid=(B // blk,),
        in_specs=specs,
        out_specs=pl.BlockSpec((blk, 1), lambda i: (i, 0)),
        out_shape=jax.ShapeDtypeStruct((B, 1), jnp.float32),
    )(*args)


def kernel(user_indices, item_indices, emb_user_mlp, emb_item_mlp,
           emb_user_mf, emb_item_mf, W0, b0, W1, b1, W_out, b_out):
    t_um = _repack(emb_user_mlp.T, 1 << _BLOG)
    t_im = _repack(emb_item_mlp.T, 1 << _BLOG)
    t_uf = _repack(emb_user_mf.T, 1 << _BLOG)
    t_if = _repack(emb_item_mf.T, 1 << _BLOG)

    xu, xi, xuf, xif = _sc_gather(
        user_indices, item_indices, t_um, t_im, t_uf, t_if)
    uidx2 = user_indices.reshape(-1, 1)
    iidx2 = item_indices.reshape(-1, 1)
    return _tc_mlp(uidx2, iidx2, xu, xi, xuf, xif,
                   W0, b0, W1, b1, W_out, b_out)


# speedup vs baseline: 3.7052x; 1.1393x over previous
import functools

import jax
import jax.numpy as jnp
from jax import lax
from jax.experimental import pallas as pl
from jax.experimental.pallas import tpu as pltpu
from jax.experimental.pallas import tpu_sc as plsc

_LANES = 128
_BLOG = 15
_QLOG_MLP = _BLOG - 2
_QLOG_MF = _BLOG - 3


def _repack_body(n, xt, out):
    d, w = xt.shape
    g = _LANES // d
    wc = w // g
    col = lax.broadcasted_iota(jnp.int32, (d, w), 1)
    x = jnp.where(col < n - pl.program_id(0) * w, xt[...], 0.0)
    lhs = jnp.concatenate([x[:, p * wc:(p + 1) * wc] for p in range(g)],
                          axis=0)
    eye = jnp.eye(_LANES, dtype=jnp.float32)
    out[...] = lax.dot_general(lhs, eye, (((0,), (0,)), ((), ())),
                               preferred_element_type=jnp.float32)


def _repack(xT, blk_n):
    d, n = xT.shape
    grid = (n + blk_n - 1) // blk_n
    rows_per_blk = blk_n * d // _LANES
    return pl.pallas_call(
        functools.partial(_repack_body, n),
        grid=(grid,),
        in_specs=[pl.BlockSpec((d, blk_n), lambda i: (0, i))],
        out_specs=pl.BlockSpec((rows_per_blk, _LANES), lambda i: (i, 0)),
        out_shape=jax.ShapeDtypeStruct((grid * rows_per_blk, _LANES),
                                       jnp.float32),
    )(xT)


def _sc_gather(user_idx, item_idx, t_um, t_im, t_uf, t_if):
    B = user_idx.shape[0]
    info = plsc.get_sparse_core_info()
    nc, ns = info.num_cores, info.num_subcores
    nw = nc * ns
    bpw = B // nw
    sub = 128
    nsub = bpw // sub

    mesh = plsc.VectorSubcoreMesh(core_axis_name="c", subcore_axis_name="s")
    out_sds = jax.ShapeDtypeStruct((B, _LANES), jnp.float32)

    @functools.partial(
        pl.kernel,
        mesh=mesh,
        out_type=[out_sds, out_sds, out_sds, out_sds],
        scratch_types=[
            pltpu.VMEM((bpw,), jnp.int32),
            pltpu.VMEM((bpw,), jnp.int32),
            pltpu.VMEM((bpw,), jnp.int32),
            pltpu.VMEM((bpw,), jnp.int32),
            pltpu.VMEM((bpw,), jnp.int32),
            pltpu.VMEM((bpw,), jnp.int32),
            pltpu.VMEM((sub, _LANES), jnp.float32),
            pltpu.VMEM((sub, _LANES), jnp.float32),
            pltpu.VMEM((sub, _LANES), jnp.float32),
            pltpu.VMEM((sub, _LANES), jnp.float32),
            pltpu.SemaphoreType.DMA,
        ],
        compiler_params=pltpu.CompilerParams(use_tc_tiling_on_sc=True),
    )
    def gather_kernel(uidx_hbm, iidx_hbm, um_hbm, im_hbm, uf_hbm, if_hbm,
                      o_um, o_im, o_uf, o_if,
                      idx_u, idx_i, q_u, q_i, q8_u, q8_i,
                      b_um, b_im, b_uf, b_if, sem):
        wid = lax.axis_index("s") * nc + lax.axis_index("c")
        base = wid * bpw
        pltpu.sync_copy(uidx_hbm.at[pl.ds(base, bpw)], idx_u)
        pltpu.sync_copy(iidx_hbm.at[pl.ds(base, bpw)], idx_i)
        for c in range(bpw // 16):
            sl = pl.ds(c * 16, 16)
            u = idx_u[sl]
            i = idx_i[sl]
            ub = lax.shift_right_logical(u, _BLOG)
            ib = lax.shift_right_logical(i, _BLOG)
            q_u[sl] = lax.shift_left(ub, _QLOG_MLP) | (u & (2**_QLOG_MLP - 1))
            q_i[sl] = lax.shift_left(ib, _QLOG_MLP) | (i & (2**_QLOG_MLP - 1))
            q8_u[sl] = lax.shift_left(ub, _QLOG_MF) | (u & (2**_QLOG_MF - 1))
            q8_i[sl] = lax.shift_left(ib, _QLOG_MF) | (i & (2**_QLOG_MF - 1))
        for s in range(nsub):
            qsl = pl.ds(s * sub, sub)
            c1 = pltpu.async_copy(um_hbm.at[q_u.at[qsl]], b_um, sem)
            c2 = pltpu.async_copy(im_hbm.at[q_i.at[qsl]], b_im, sem)
            c3 = pltpu.async_copy(uf_hbm.at[q8_u.at[qsl]], b_uf, sem)
            c4 = pltpu.async_copy(if_hbm.at[q8_i.at[qsl]], b_if, sem)
            c1.wait()
            c2.wait()
            c3.wait()
            c4.wait()
            osl = pl.ds(base + s * sub, sub)
            pltpu.sync_copy(b_um, o_um.at[osl])
            pltpu.sync_copy(b_im, o_im.at[osl])
            pltpu.sync_copy(b_uf, o_uf.at[osl])
            pltpu.sync_copy(b_if, o_if.at[osl])

    return gather_kernel(user_idx, item_idx, t_um, t_im, t_uf, t_if)


def _mlp_body(uidx, iidx, xu, xi, xuf, xif,
              w0u, w0i, b0, w1, b1, w2, wu, wi, bo, out):
    pu = lax.shift_right_logical(uidx[...], _QLOG_MLP) & 3
    pi = lax.shift_right_logical(iidx[...], _QLOG_MLP) & 3
    pu8 = lax.shift_right_logical(uidx[...], _QLOG_MF) & 7
    pi8 = lax.shift_right_logical(iidx[...], _QLOG_MF) & 7

    yu = xu[...] @ w0u[...]
    yi = xi[...] @ w0i[...]
    d = w1.shape[0]
    h = jnp.zeros_like(yu[:, :d])
    for p in range(4):
        h = (h + jnp.where(pu == p, yu[:, p * d:(p + 1) * d], 0.0)
             + jnp.where(pi == p, yi[:, p * d:(p + 1) * d], 0.0))
    h = jnp.maximum(h + b0[...], 0.0)
    h = jnp.maximum(h @ w1[...] + b1[...], 0.0)

    zu = xuf[...] @ wu[...]
    zi = xif[...] @ wi[...]
    iota8 = lax.broadcasted_iota(jnp.int32, zu.shape, 1)
    zu = jnp.sum(jnp.where(iota8 == pu8, zu, 0.0), axis=1, keepdims=True)
    zi = jnp.sum(jnp.where(iota8 == pi8, zi, 0.0), axis=1, keepdims=True)

    logit = h @ w2[...] + zu + zi + bo[...]
    out[...] = jax.nn.sigmoid(logit)


def _tc_mlp(uidx2, iidx2, xu, xi, xuf, xif, W0, b0, W1, b1, W_out, b_out):
    B = xu.shape[0]
    d_mlp = W0.shape[0] // 2
    h1 = W0.shape[1]
    h2 = W1.shape[1]
    d_mf = (W_out.shape[0] - h2) // 2
    blk = 4096

    eye4 = jnp.eye(4, dtype=jnp.float32)
    eye8 = jnp.eye(8, dtype=jnp.float32)
    w0u = jnp.kron(eye4, W0[:d_mlp])
    w0i = jnp.kron(eye4, W0[d_mlp:])
    wu = jnp.kron(eye8, W_out[h2:h2 + d_mf])
    wi = jnp.kron(eye8, W_out[h2 + d_mf:])
    w2 = W_out[:h2]
    b0r = b0.reshape(1, h1)
    b1r = b1.reshape(1, h2)
    bor = b_out.reshape(1, 1)

    full = lambda a: pl.BlockSpec(a.shape, lambda i: (0,) * a.ndim)
    bspec = lambda d: pl.BlockSpec((blk, d), lambda i: (i, 0))

    args = (uidx2, iidx2, xu, xi, xuf, xif,
            w0u, w0i, b0r, W1, b1r, w2, wu, wi, bor)
    specs = [bspec(1), bspec(1), bspec(_LANES), bspec(_LANES),
             bspec(_LANES), bspec(_LANES)] + [full(a) for a in args[6:]]

    return pl.pallas_call(
        _mlp_body,
        grid=(B // blk,),
        in_specs=specs,
        out_specs=pl.BlockSpec((blk, 1), lambda i: (i, 0)),
        out_shape=jax.ShapeDtypeStruct((B, 1), jnp.float32),
    )(*args)


def kernel(user_indices, item_indices, emb_user_mlp, emb_item_mlp,
           emb_user_mf, emb_item_mf, W0, b0, W1, b1, W_out, b_out):
    t_um = _repack(emb_user_mlp.T, 1 << _BLOG)
    t_im = _repack(emb_item_mlp.T, 1 << _BLOG)
    t_uf = _repack(emb_user_mf.T, 1 << _BLOG)
    t_if = _repack(emb_item_mf.T, 1 << _BLOG)

    xu, xi, xuf, xif = _sc_gather(
        user_indices, item_indices, t_um, t_im, t_uf, t_if)
    uidx2 = user_indices.reshape(-1, 1)
    iidx2 = item_indices.reshape(-1, 1)
    return _tc_mlp(uidx2, iidx2, xu, xi, xuf, xif,
                   W0, b0, W1, b1, W_out, b_out)


# speedup vs baseline: 3.8600x; 1.0418x over previous
import functools

import jax
import jax.numpy as jnp
from jax import lax
from jax.experimental import pallas as pl
from jax.experimental.pallas import tpu as pltpu
from jax.experimental.pallas import tpu_sc as plsc

_LANES = 128
_BLOG = 16
_QLOG_MLP = _BLOG - 2
_QLOG_MF = _BLOG - 3


def _repack_body(n, xt, out):
    d, w = xt.shape
    g = _LANES // d
    wc = w // g
    col = lax.broadcasted_iota(jnp.int32, (d, w), 1)
    x = jnp.where(col < n - pl.program_id(0) * w, xt[...], 0.0)
    lhs = jnp.concatenate([x[:, p * wc:(p + 1) * wc] for p in range(g)],
                          axis=0)
    eye = jnp.eye(_LANES, dtype=jnp.float32)
    out[...] = lax.dot_general(lhs, eye, (((0,), (0,)), ((), ())),
                               preferred_element_type=jnp.float32)


def _repack(xT, blk_n):
    d, n = xT.shape
    grid = (n + blk_n - 1) // blk_n
    rows_per_blk = blk_n * d // _LANES
    return pl.pallas_call(
        functools.partial(_repack_body, n),
        grid=(grid,),
        in_specs=[pl.BlockSpec((d, blk_n), lambda i: (0, i))],
        out_specs=pl.BlockSpec((rows_per_blk, _LANES), lambda i: (i, 0)),
        out_shape=jax.ShapeDtypeStruct((grid * rows_per_blk, _LANES),
                                       jnp.float32),
    )(xT)


def _sc_gather(user_idx, item_idx, t_um, t_im, t_uf, t_if):
    B = user_idx.shape[0]
    info = plsc.get_sparse_core_info()
    nc, ns = info.num_cores, info.num_subcores
    nw = nc * ns
    bpw = B // nw
    sub = 128
    nsub = bpw // sub

    mesh = plsc.VectorSubcoreMesh(core_axis_name="c", subcore_axis_name="s")
    out_sds = jax.ShapeDtypeStruct((B, _LANES), jnp.float32)

    @functools.partial(
        pl.kernel,
        mesh=mesh,
        out_type=[out_sds, out_sds, out_sds, out_sds],
        scratch_types=[
            pltpu.VMEM((bpw,), jnp.int32),
            pltpu.VMEM((bpw,), jnp.int32),
            pltpu.VMEM((bpw,), jnp.int32),
            pltpu.VMEM((bpw,), jnp.int32),
            pltpu.VMEM((bpw,), jnp.int32),
            pltpu.VMEM((bpw,), jnp.int32),
            pltpu.VMEM((sub, _LANES), jnp.float32),
            pltpu.VMEM((sub, _LANES), jnp.float32),
            pltpu.VMEM((sub, _LANES), jnp.float32),
            pltpu.VMEM((sub, _LANES), jnp.float32),
            pltpu.SemaphoreType.DMA,
        ],
        compiler_params=pltpu.CompilerParams(use_tc_tiling_on_sc=True),
    )
    def gather_kernel(uidx_hbm, iidx_hbm, um_hbm, im_hbm, uf_hbm, if_hbm,
                      o_um, o_im, o_uf, o_if,
                      idx_u, idx_i, q_u, q_i, q8_u, q8_i,
                      b_um, b_im, b_uf, b_if, sem):
        wid = lax.axis_index("s") * nc + lax.axis_index("c")
        base = wid * bpw
        pltpu.sync_copy(uidx_hbm.at[pl.ds(base, bpw)], idx_u)
        pltpu.sync_copy(iidx_hbm.at[pl.ds(base, bpw)], idx_i)
        for c in range(bpw // 16):
            sl = pl.ds(c * 16, 16)
            u = idx_u[sl]
            i = idx_i[sl]
            ub = lax.shift_right_logical(u, _BLOG)
            ib = lax.shift_right_logical(i, _BLOG)
            q_u[sl] = lax.shift_left(ub, _QLOG_MLP) | (u & (2**_QLOG_MLP - 1))
            q_i[sl] = lax.shift_left(ib, _QLOG_MLP) | (i & (2**_QLOG_MLP - 1))
            q8_u[sl] = lax.shift_left(ub, _QLOG_MF) | (u & (2**_QLOG_MF - 1))
            q8_i[sl] = lax.shift_left(ib, _QLOG_MF) | (i & (2**_QLOG_MF - 1))
        for s in range(nsub):
            qsl = pl.ds(s * sub, sub)
            c1 = pltpu.async_copy(um_hbm.at[q_u.at[qsl]], b_um, sem)
            c2 = pltpu.async_copy(im_hbm.at[q_i.at[qsl]], b_im, sem)
            c3 = pltpu.async_copy(uf_hbm.at[q8_u.at[qsl]], b_uf, sem)
            c4 = pltpu.async_copy(if_hbm.at[q8_i.at[qsl]], b_if, sem)
            c1.wait()
            c2.wait()
            c3.wait()
            c4.wait()
            osl = pl.ds(base + s * sub, sub)
            pltpu.sync_copy(b_um, o_um.at[osl])
            pltpu.sync_copy(b_im, o_im.at[osl])
            pltpu.sync_copy(b_uf, o_uf.at[osl])
            pltpu.sync_copy(b_if, o_if.at[osl])

    return gather_kernel(user_idx, item_idx, t_um, t_im, t_uf, t_if)


def _mlp_body(uidx, iidx, xu, xi, xuf, xif,
              w0u, w0i, b0, w1, b1, w2, wu, wi, bo, out):
    pu = lax.shift_right_logical(uidx[...], _QLOG_MLP) & 3
    pi = lax.shift_right_logical(iidx[...], _QLOG_MLP) & 3
    pu8 = lax.shift_right_logical(uidx[...], _QLOG_MF) & 7
    pi8 = lax.shift_right_logical(iidx[...], _QLOG_MF) & 7

    yu = xu[...] @ w0u[...]
    yi = xi[...] @ w0i[...]
    d = w1.shape[0]
    h = jnp.zeros_like(yu[:, :d])
    for p in range(4):
        h = (h + jnp.where(pu == p, yu[:, p * d:(p + 1) * d], 0.0)
             + jnp.where(pi == p, yi[:, p * d:(p + 1) * d], 0.0))
    h = jnp.maximum(h + b0[...], 0.0)
    h = jnp.maximum(h @ w1[...] + b1[...], 0.0)

    zu = xuf[...] @ wu[...]
    zi = xif[...] @ wi[...]
    iota8 = lax.broadcasted_iota(jnp.int32, zu.shape, 1)
    zu = jnp.sum(jnp.where(iota8 == pu8, zu, 0.0), axis=1, keepdims=True)
    zi = jnp.sum(jnp.where(iota8 == pi8, zi, 0.0), axis=1, keepdims=True)

    logit = h @ w2[...] + zu + zi + bo[...]
    out[...] = jax.nn.sigmoid(logit)


def _tc_mlp(uidx2, iidx2, xu, xi, xuf, xif, W0, b0, W1, b1, W_out, b_out):
    B = xu.shape[0]
    d_mlp = W0.shape[0] // 2
    h1 = W0.shape[1]
    h2 = W1.shape[1]
    d_mf = (W_out.shape[0] - h2) // 2
    blk = 4096

    eye4 = jnp.eye(4, dtype=jnp.float32)
    eye8 = jnp.eye(8, dtype=jnp.float32)
    w0u = jnp.kron(eye4, W0[:d_mlp])
    w0i = jnp.kron(eye4, W0[d_mlp:])
    wu = jnp.kron(eye8, W_out[h2:h2 + d_mf])
    wi = jnp.kron(eye8, W_out[h2 + d_mf:])
    w2 = W_out[:h2]
    b0r = b0.reshape(1, h1)
    b1r = b1.reshape(1, h2)
    bor = b_out.reshape(1, 1)

    full = lambda a: pl.BlockSpec(a.shape, lambda i: (0,) * a.ndim)
    bspec = lambda d: pl.BlockSpec((blk, d), lambda i: (i, 0))

    args = (uidx2, iidx2, xu, xi, xuf, xif,
            w0u, w0i, b0r, W1, b1r, w2, wu, wi, bor)
    specs = [bspec(1), bspec(1), bspec(_LANES), bspec(_LANES),
             bspec(_LANES), bspec(_LANES)] + [full(a) for a in args[6:]]

    return pl.pallas_call(
        _mlp_body,
        grid=(B // blk,),
        in_specs=specs,
        out_specs=pl.BlockSpec((blk, 1), lambda i: (i, 0)),
        out_shape=jax.ShapeDtypeStruct((B, 1), jnp.float32),
    )(*args)


def kernel(user_indices, item_indices, emb_user_mlp, emb_item_mlp,
           emb_user_mf, emb_item_mf, W0, b0, W1, b1, W_out, b_out):
    t_um = _repack(emb_user_mlp.T, 1 << _BLOG)
    t_im = _repack(emb_item_mlp.T, 1 << _BLOG)
    t_uf = _repack(emb_user_mf.T, 1 << _BLOG)
    t_if = _repack(emb_item_mf.T, 1 << _BLOG)

    xu, xi, xuf, xif = _sc_gather(
        user_indices, item_indices, t_um, t_im, t_uf, t_if)
    uidx2 = user_indices.reshape(-1, 1)
    iidx2 = item_indices.reshape(-1, 1)
    return _tc_mlp(uidx2, iidx2, xu, xi, xuf, xif,
                   W0, b0, W1, b1, W_out, b_out)


# speedup vs baseline: 3.8726x; 1.0033x over previous
import functools

import jax
import jax.numpy as jnp
from jax import lax
from jax.experimental import pallas as pl
from jax.experimental.pallas import tpu as pltpu
from jax.experimental.pallas import tpu_sc as plsc

_LANES = 128
_BLOG = 16
_QLOG_MLP = _BLOG - 2
_QLOG_MF = _BLOG - 3


def _repack_body(n, xt, out):
    d, w = xt.shape
    g = _LANES // d
    wc = w // g
    col = lax.broadcasted_iota(jnp.int32, (d, w), 1)
    x = jnp.where(col < n - pl.program_id(0) * w, xt[...], 0.0)
    lhs = jnp.concatenate([x[:, p * wc:(p + 1) * wc] for p in range(g)],
                          axis=0)
    eye = jnp.eye(_LANES, dtype=jnp.float32)
    out[...] = lax.dot_general(lhs, eye, (((0,), (0,)), ((), ())),
                               preferred_element_type=jnp.float32)


def _repack(xT, blk_n):
    d, n = xT.shape
    grid = (n + blk_n - 1) // blk_n
    rows_per_blk = blk_n * d // _LANES
    return pl.pallas_call(
        functools.partial(_repack_body, n),
        grid=(grid,),
        in_specs=[pl.BlockSpec((d, blk_n), lambda i: (0, i))],
        out_specs=pl.BlockSpec((rows_per_blk, _LANES), lambda i: (i, 0)),
        out_shape=jax.ShapeDtypeStruct((grid * rows_per_blk, _LANES),
                                       jnp.float32),
    )(xT)


def _sc_gather(user_idx, item_idx, t_um, t_im, t_uf, t_if):
    B = user_idx.shape[0]
    info = plsc.get_sparse_core_info()
    nc, ns = info.num_cores, info.num_subcores
    nw = nc * ns
    bpw = B // nw
    sub = 64
    nsub = bpw // sub

    mesh = plsc.VectorSubcoreMesh(core_axis_name="c", subcore_axis_name="s")
    out_sds = jax.ShapeDtypeStruct((B, _LANES), jnp.float32)

    @functools.partial(
        pl.kernel,
        mesh=mesh,
        out_type=[out_sds, out_sds, out_sds, out_sds],
        scratch_types=[
            pltpu.VMEM((bpw,), jnp.int32),
            pltpu.VMEM((bpw,), jnp.int32),
            pltpu.VMEM((bpw,), jnp.int32),
            pltpu.VMEM((bpw,), jnp.int32),
            pltpu.VMEM((bpw,), jnp.int32),
            pltpu.VMEM((bpw,), jnp.int32),
            pltpu.VMEM((2, sub, _LANES), jnp.float32),
            pltpu.VMEM((2, sub, _LANES), jnp.float32),
            pltpu.VMEM((2, sub, _LANES), jnp.float32),
            pltpu.VMEM((2, sub, _LANES), jnp.float32),
            pltpu.SemaphoreType.DMA,
        ],
        compiler_params=pltpu.CompilerParams(use_tc_tiling_on_sc=True),
    )
    def gather_kernel(uidx_hbm, iidx_hbm, um_hbm, im_hbm, uf_hbm, if_hbm,
                      o_um, o_im, o_uf, o_if,
                      idx_u, idx_i, q_u, q_i, q8_u, q8_i,
                      b_um, b_im, b_uf, b_if, sem):
        wid = lax.axis_index("s") * nc + lax.axis_index("c")
        base = wid * bpw
        pltpu.sync_copy(uidx_hbm.at[pl.ds(base, bpw)], idx_u)
        pltpu.sync_copy(iidx_hbm.at[pl.ds(base, bpw)], idx_i)
        for c in range(bpw // 16):
            sl = pl.ds(c * 16, 16)
            u = idx_u[sl]
            i = idx_i[sl]
            ub = lax.shift_right_logical(u, _BLOG)
            ib = lax.shift_right_logical(i, _BLOG)
            q_u[sl] = lax.shift_left(ub, _QLOG_MLP) | (u & (2**_QLOG_MLP - 1))
            q_i[sl] = lax.shift_left(ib, _QLOG_MLP) | (i & (2**_QLOG_MLP - 1))
            q8_u[sl] = lax.shift_left(ub, _QLOG_MF) | (u & (2**_QLOG_MF - 1))
            q8_i[sl] = lax.shift_left(ib, _QLOG_MF) | (i & (2**_QLOG_MF - 1))
        def drain(s, cps):
            for c in cps:
                c.wait()
            k = s % 2
            osl = pl.ds(base + s * sub, sub)
            pltpu.sync_copy(b_um.at[k], o_um.at[osl])
            pltpu.sync_copy(b_im.at[k], o_im.at[osl])
            pltpu.sync_copy(b_uf.at[k], o_uf.at[osl])
            pltpu.sync_copy(b_if.at[k], o_if.at[osl])

        prev = None
        for s in range(nsub):
            k = s % 2
            qsl = pl.ds(s * sub, sub)
            cps = [
                pltpu.async_copy(um_hbm.at[q_u.at[qsl]], b_um.at[k], sem),
                pltpu.async_copy(im_hbm.at[q_i.at[qsl]], b_im.at[k], sem),
                pltpu.async_copy(uf_hbm.at[q8_u.at[qsl]], b_uf.at[k], sem),
                pltpu.async_copy(if_hbm.at[q8_i.at[qsl]], b_if.at[k], sem),
            ]
            if prev is not None:
                drain(*prev)
            prev = (s, cps)
        drain(*prev)

    return gather_kernel(user_idx, item_idx, t_um, t_im, t_uf, t_if)


def _mlp_body(uidx, iidx, xu, xi, xuf, xif,
              w0u, w0i, b0, w1, b1, w2, wu, wi, bo, out):
    pu = lax.shift_right_logical(uidx[...], _QLOG_MLP) & 3
    pi = lax.shift_right_logical(iidx[...], _QLOG_MLP) & 3
    pu8 = lax.shift_right_logical(uidx[...], _QLOG_MF) & 7
    pi8 = lax.shift_right_logical(iidx[...], _QLOG_MF) & 7

    yu = xu[...] @ w0u[...]
    yi = xi[...] @ w0i[...]
    d = w1.shape[0]
    h = jnp.zeros_like(yu[:, :d])
    for p in range(4):
        h = (h + jnp.where(pu == p, yu[:, p * d:(p + 1) * d], 0.0)
             + jnp.where(pi == p, yi[:, p * d:(p + 1) * d], 0.0))
    h = jnp.maximum(h + b0[...], 0.0)
    h = jnp.maximum(h @ w1[...] + b1[...], 0.0)

    zu = xuf[...] @ wu[...]
    zi = xif[...] @ wi[...]
    iota8 = lax.broadcasted_iota(jnp.int32, zu.shape, 1)
    zu = jnp.sum(jnp.where(iota8 == pu8, zu, 0.0), axis=1, keepdims=True)
    zi = jnp.sum(jnp.where(iota8 == pi8, zi, 0.0), axis=1, keepdims=True)

    logit = h @ w2[...] + zu + zi + bo[...]
    out[...] = jax.nn.sigmoid(logit)


def _tc_mlp(uidx2, iidx2, xu, xi, xuf, xif, W0, b0, W1, b1, W_out, b_out):
    B = xu.shape[0]
    d_mlp = W0.shape[0] // 2
    h1 = W0.shape[1]
    h2 = W1.shape[1]
    d_mf = (W_out.shape[0] - h2) // 2
    blk = 4096

    eye4 = jnp.eye(4, dtype=jnp.float32)
    eye8 = jnp.eye(8, dtype=jnp.float32)
    w0u = jnp.kron(eye4, W0[:d_mlp])
    w0i = jnp.kron(eye4, W0[d_mlp:])
    wu = jnp.kron(eye8, W_out[h2:h2 + d_mf])
    wi = jnp.kron(eye8, W_out[h2 + d_mf:])
    w2 = W_out[:h2]
    b0r = b0.reshape(1, h1)
    b1r = b1.reshape(1, h2)
    bor = b_out.reshape(1, 1)

    full = lambda a: pl.BlockSpec(a.shape, lambda i: (0,) * a.ndim)
    bspec = lambda d: pl.BlockSpec((blk, d), lambda i: (i, 0))

    args = (uidx2, iidx2, xu, xi, xuf, xif,
            w0u, w0i, b0r, W1, b1r, w2, wu, wi, bor)
    specs = [bspec(1), bspec(1), bspec(_LANES), bspec(_LANES),
             bspec(_LANES), bspec(_LANES)] + [full(a) for a in args[6:]]

    return pl.pallas_call(
        _mlp_body,
        grid=(B // blk,),
        in_specs=specs,
        out_specs=pl.BlockSpec((blk, 1), lambda i: (i, 0)),
        out_shape=jax.ShapeDtypeStruct((B, 1), jnp.float32),
    )(*args)


def kernel(user_indices, item_indices, emb_user_mlp, emb_item_mlp,
           emb_user_mf, emb_item_mf, W0, b0, W1, b1, W_out, b_out):
    t_um = _repack(emb_user_mlp.T, 1 << _BLOG)
    t_im = _repack(emb_item_mlp.T, 1 << _BLOG)
    t_uf = _repack(emb_user_mf.T, 1 << _BLOG)
    t_if = _repack(emb_item_mf.T, 1 << _BLOG)

    xu, xi, xuf, xif = _sc_gather(
        user_indices, item_indices, t_um, t_im, t_uf, t_if)
    uidx2 = user_indices.reshape(-1, 1)
    iidx2 = item_indices.reshape(-1, 1)
    return _tc_mlp(uidx2, iidx2, xu, xi, xuf, xif,
                   W0, b0, W1, b1, W_out, b_out)
